# baseline (speedup 1.0000x reference)
import functools

import jax
import jax.numpy as jnp
from jax import lax
from jax.experimental import pallas as pl
from jax.experimental.pallas import tpu as pltpu
from jax.experimental.pallas import tpu_sc as plsc

_N = 100000
_R = 8
_D = 32
_C = 2000

_E = 1600000
_H = _N // 2
_NTILES = 16
_CHUNK = 256
_EPT = 100352
_EPAD = _EPT * _NTILES
_TPH = 3128
_ZR = 1024


def _dense_body(s_ref, cnt_ref, x_ref, w_ref, root_ref, b_ref, o_ref, *, relu):
    acc = jnp.dot(x_ref[...], root_ref[...], preferred_element_type=jnp.float32)
    for r in range(_R):
        inv = 1.0 / jnp.maximum(cnt_ref[r, 0, 0, :], 1.0)
        acc += jnp.dot(s_ref[r] * inv[:, None], w_ref[r],
                       preferred_element_type=jnp.float32)
    acc += b_ref[...]
    o_ref[...] = jnp.maximum(acc, 0.0) if relu else acc


def _dense_phase(s3, cnt2, x, w, root, b, relu):
    grid = (_N // _C,)
    return pl.pallas_call(
        functools.partial(_dense_body, relu=relu),
        grid=grid,
        in_specs=[
            pl.BlockSpec((_R, _C, _D), lambda i: (0, i, 0)),
            pl.BlockSpec((_R, 1, 1, _C), lambda i: (0, i, 0, 0)),
            pl.BlockSpec((_C, _D), lambda i: (i, 0)),
            pl.BlockSpec((_R, _D, _D), lambda i: (0, 0, 0)),
            pl.BlockSpec((_D, _D), lambda i: (0, 0)),
            pl.BlockSpec((1, _D), lambda i: (0, 0)),
        ],
        out_specs=pl.BlockSpec((_C, _D), lambda i: (i, 0)),
        out_shape=jax.ShapeDtypeStruct((_N, _D), jnp.float32),
    )(s3, cnt2.reshape(_R, _N // _C, 1, _C), x, w, root, b.reshape(1, _D))


def _sc_agg_body(xtab, srcs, dsts, ets, s_out,
                 acc, src_v, dst_v, et_v, idx2_v, rows_v, sem):
    c = lax.axis_index("c")
    s = lax.axis_index("s")
    cbase = c * _H

    def _zrow(i, _):
        rows_v[i, 0:16] = jnp.zeros((16,), jnp.float32)
        rows_v[i, 16:32] = jnp.zeros((16,), jnp.float32)
        return 0

    def _chunk(i, r):
        off = s * _EPT + i * _CHUNK
        pltpu.sync_copy(srcs.at[pl.ds(off, _CHUNK)], src_v)
        pltpu.sync_copy(dsts.at[pl.ds(off, _CHUNK)], dst_v)
        pltpu.sync_copy(ets.at[pl.ds(off, _CHUNK)], et_v)
        pltpu.async_copy(xtab.at[src_v], rows_v, sem).wait()

        def _mkidx(j, r):
            def _vreg(k, r):
                dv = dst_v[pl.ds((j * 8 + k) * 16, 16)]
                ev = et_v[pl.ds((j * 8 + k) * 16, 16)]
                loc = dv - cbase
                m = (ev == r) & (dv >= cbase) & (loc < _H)
                idx2_v[j, pl.ds(k * 16, 16)] = jnp.where(m, loc, _H + s)
                return r
            return lax.fori_loop(0, 8, _vreg, r)
        lax.fori_loop(0, _CHUNK // 128, _mkidx, r)

        def _scat(j, r):
            pltpu.sync_copy(rows_v.at[pl.ds(j * 128, 128)],
                            acc.at[idx2_v.at[j]], add=True)
            return r
        lax.fori_loop(0, _CHUNK // 128, _scat, r)
        return r

    zbase = s * _TPH
    for r in range(_R):
        lax.fori_loop(0, _CHUNK, _zrow, 0)
        for k in range(12):
            pltpu.sync_copy(rows_v, acc.at[pl.ds(zbase + k * _CHUNK, _CHUNK)])

        @pl.when(s < _NTILES - 1)
        def _():
            pltpu.sync_copy(rows_v.at[pl.ds(0, 56)],
                            acc.at[pl.ds(zbase + 12 * _CHUNK, 56)])

        @pl.when(s == _NTILES - 1)
        def _():
            pltpu.sync_copy(rows_v.at[pl.ds(0, 24)],
                            acc.at[pl.ds(zbase + 12 * _CHUNK, 24)])

        plsc.subcore_barrier()
        lax.fori_loop(0, _EPT // _CHUNK, _chunk, jnp.int32(r))
        plsc.subcore_barrier()

        @pl.when(s < _NTILES - 1)
        def _():
            pltpu.sync_copy(acc.at[pl.ds(zbase, _TPH)],
                            s_out.at[pl.ds(r * _N + cbase + zbase, _TPH)])

        @pl.when(s == _NTILES - 1)
        def _():
            pltpu.sync_copy(acc.at[pl.ds(zbase, 3080)],
                            s_out.at[pl.ds(r * _N + cbase + zbase, 3080)])


def _sc_aggregate(xtab, srcs, dsts, ets):
    mesh = plsc.VectorSubcoreMesh(core_axis_name="c", subcore_axis_name="s")
    return pl.kernel(
        _sc_agg_body,
        out_type=jax.ShapeDtypeStruct((_R * _N, _D), jnp.float32),
        mesh=mesh,
        compiler_params=pltpu.CompilerParams(use_tc_tiling_on_sc=False),
        scratch_types=[
            pltpu.VMEM_SHARED((_H + _NTILES, _D), jnp.float32),
            pltpu.VMEM((_CHUNK,), jnp.int32),
            pltpu.VMEM((_CHUNK,), jnp.int32),
            pltpu.VMEM((_CHUNK,), jnp.int32),
            pltpu.VMEM((_CHUNK // 128, 128), jnp.int32),
            pltpu.VMEM((_CHUNK, _D), jnp.float32),
            pltpu.SemaphoreType.DMA,
        ],
    )(xtab, srcs, dsts, ets)


def kernel(x, edge_index, edge_type, W1, root1, b1, W2, root2, b2):
    src = edge_index[0]
    dst = edge_index[1]
    seg = edge_type * _N + dst

    cnt = jax.ops.segment_sum(jnp.ones((src.shape[0],), jnp.float32), seg,
                              num_segments=_N * _R)
    cnt2 = cnt.reshape(_R, _N)

    pad = _EPAD - _E
    srcs = jnp.concatenate([src, jnp.zeros((pad,), jnp.int32)])
    dsts = jnp.concatenate([dst, jnp.zeros((pad,), jnp.int32)])
    ets = jnp.concatenate([edge_type, jnp.full((pad,), _R, jnp.int32)])

    h = x
    for (w, root, b, relu) in ((W1, root1, b1, True), (W2, root2, b2, False)):
        s = _sc_aggregate(h, srcs, dsts, ets)
        h = _dense_phase(s.reshape(_R, _N, _D), cnt2, h, w, root, b, relu)
    return h


# speedup vs baseline: 1.4369x; 1.4369x over previous
import functools

import jax
import jax.numpy as jnp
from jax import lax
from jax.experimental import pallas as pl
from jax.experimental.pallas import tpu as pltpu
from jax.experimental.pallas import tpu_sc as plsc

_N = 100000
_R = 8
_D = 32
_C = 2000

_E = 1600000
_H = _N // 2
_NTILES = 16
_CHUNK = 256
_EPT = 100352
_EPAD = _EPT * _NTILES
_TPH = 3128
_ZR = 1024


def _dense_body(s_ref, cnt_ref, x_ref, w_ref, root_ref, b_ref, o_ref, *, relu):
    acc = jnp.dot(x_ref[...], root_ref[...], preferred_element_type=jnp.float32)
    for r in range(_R):
        inv = 1.0 / jnp.maximum(cnt_ref[r, 0, 0, :], 1.0)
        acc += jnp.dot(s_ref[r] * inv[:, None], w_ref[r],
                       preferred_element_type=jnp.float32)
    acc += b_ref[...]
    o_ref[...] = jnp.maximum(acc, 0.0) if relu else acc


def _dense_phase(s3, cnt2, x, w, root, b, relu):
    grid = (_N // _C,)
    return pl.pallas_call(
        functools.partial(_dense_body, relu=relu),
        grid=grid,
        in_specs=[
            pl.BlockSpec((_R, _C, _D), lambda i: (0, i, 0)),
            pl.BlockSpec((_R, 1, 1, _C), lambda i: (0, i, 0, 0)),
            pl.BlockSpec((_C, _D), lambda i: (i, 0)),
            pl.BlockSpec((_R, _D, _D), lambda i: (0, 0, 0)),
            pl.BlockSpec((_D, _D), lambda i: (0, 0)),
            pl.BlockSpec((1, _D), lambda i: (0, 0)),
        ],
        out_specs=pl.BlockSpec((_C, _D), lambda i: (i, 0)),
        out_shape=jax.ShapeDtypeStruct((_N, _D), jnp.float32),
    )(s3, cnt2.reshape(_R, _N // _C, 1, _C), x, w, root, b.reshape(1, _D))


def _sc_agg_body(xtab, srcs, segs, s_out,
                 acc, src0, src1, seg0, seg1, idx0, idx1, rows0, rows1,
                 sem0, sem1):
    c = lax.axis_index("c")
    s = lax.axis_index("s")
    cbase = c * _H

    def _zrow(i, _):
        rows0[i, 0:16] = jnp.zeros((16,), jnp.float32)
        rows0[i, 16:32] = jnp.zeros((16,), jnp.float32)
        return 0

    def _lg(i, src_b, seg_b, rows_b, sem):
        off = s * _EPT + i * _CHUNK
        pltpu.sync_copy(srcs.at[pl.ds(off, _CHUNK)], src_b)
        pltpu.sync_copy(segs.at[pl.ds(off, _CHUNK)], seg_b)
        return pltpu.async_copy(xtab.at[src_b], rows_b, sem)

    def _mkidx(seg_b, idx_b, r):
        base = r * _N + cbase

        def _j(j, r):
            def _v(kk, r):
                loc = seg_b[pl.ds((j * 8 + kk) * 16, 16)] - base
                m = (loc >= 0) & (loc < _H)
                idx_b[j, pl.ds(kk * 16, 16)] = jnp.where(m, loc, _H + s)
                return r
            return lax.fori_loop(0, 8, _v, r)
        lax.fori_loop(0, _CHUNK // 128, _j, r)

    def _scat(rows_b, idx_b, r):
        def _j(j, r):
            pltpu.sync_copy(rows_b.at[pl.ds(j * 128, 128)],
                            acc.at[idx_b.at[j]], add=True)
            return r
        lax.fori_loop(0, _CHUNK // 128, _j, r)

    def _pair(p, r):
        d0 = _lg(2 * p, src0, seg0, rows0, sem0)
        d1 = _lg(2 * p + 1, src1, seg1, rows1, sem1)
        _mkidx(seg0, idx0, r)
        _mkidx(seg1, idx1, r)
        d0.wait()
        _scat(rows0, idx0, r)
        d1.wait()
        _scat(rows1, idx1, r)
        return r

    zbase = s * _TPH
    for r in range(_R):
        lax.fori_loop(0, _CHUNK, _zrow, 0)
        for k in range(12):
            pltpu.sync_copy(rows0, acc.at[pl.ds(zbase + k * _CHUNK, _CHUNK)])

        @pl.when(s < _NTILES - 1)
        def _():
            pltpu.sync_copy(rows0.at[pl.ds(0, 56)],
                            acc.at[pl.ds(zbase + 12 * _CHUNK, 56)])

        @pl.when(s == _NTILES - 1)
        def _():
            pltpu.sync_copy(rows0.at[pl.ds(0, 24)],
                            acc.at[pl.ds(zbase + 12 * _CHUNK, 24)])

        plsc.subcore_barrier()
        lax.fori_loop(0, _EPT // (2 * _CHUNK), _pair, jnp.int32(r))
        plsc.subcore_barrier()

        @pl.when(s < _NTILES - 1)
        def _():
            pltpu.sync_copy(acc.at[pl.ds(zbase, _TPH)],
                            s_out.at[pl.ds(r * _N + cbase + zbase, _TPH)])

        @pl.when(s == _NTILES - 1)
        def _():
            pltpu.sync_copy(acc.at[pl.ds(zbase, 3080)],
                            s_out.at[pl.ds(r * _N + cbase + zbase, 3080)])


def _sc_aggregate(xtab, srcs, segs):
    mesh = plsc.VectorSubcoreMesh(core_axis_name="c", subcore_axis_name="s")
    return pl.kernel(
        _sc_agg_body,
        out_type=jax.ShapeDtypeStruct((_R * _N, _D), jnp.float32),
        mesh=mesh,
        compiler_params=pltpu.CompilerParams(use_tc_tiling_on_sc=False),
        scratch_types=[
            pltpu.VMEM_SHARED((_H + _NTILES, _D), jnp.float32),
            pltpu.VMEM((_CHUNK,), jnp.int32),
            pltpu.VMEM((_CHUNK,), jnp.int32),
            pltpu.VMEM((_CHUNK,), jnp.int32),
            pltpu.VMEM((_CHUNK,), jnp.int32),
            pltpu.VMEM((_CHUNK // 128, 128), jnp.int32),
            pltpu.VMEM((_CHUNK // 128, 128), jnp.int32),
            pltpu.VMEM((_CHUNK, _D), jnp.float32),
            pltpu.VMEM((_CHUNK, _D), jnp.float32),
            pltpu.SemaphoreType.DMA,
            pltpu.SemaphoreType.DMA,
        ],
    )(xtab, srcs, segs)


def kernel(x, edge_index, edge_type, W1, root1, b1, W2, root2, b2):
    src = edge_index[0]
    dst = edge_index[1]
    seg = edge_type * _N + dst

    cnt = jax.ops.segment_sum(jnp.ones((src.shape[0],), jnp.float32), seg,
                              num_segments=_N * _R)
    cnt2 = cnt.reshape(_R, _N)

    pad = _EPAD - _E
    srcs = jnp.concatenate([src, jnp.zeros((pad,), jnp.int32)])
    segs = jnp.concatenate([seg, jnp.full((pad,), _R * _N, jnp.int32)])

    h = x
    for (w, root, b, relu) in ((W1, root1, b1, True), (W2, root2, b2, False)):
        s = _sc_aggregate(h, srcs, segs)
        h = _dense_phase(s.reshape(_R, _N, _D), cnt2, h, w, root, b, relu)
    return h


# speedup vs baseline: 1.4407x; 1.0027x over previous
import functools

import jax
import jax.numpy as jnp
from jax import lax
from jax.experimental import pallas as pl
from jax.experimental.pallas import tpu as pltpu
from jax.experimental.pallas import tpu_sc as plsc

_N = 100000
_R = 8
_D = 32
_C = 2000

_E = 1600000
_H = _N // 2
_NTILES = 16
_CHUNK = 256
_EPT = 100352
_EPAD = _EPT * _NTILES
_TPH = 3128
_ZR = 1024


def _dense_body(s_ref, cnt_ref, x_ref, w_ref, root_ref, b_ref, o_ref, *, relu):
    acc = jnp.dot(x_ref[...], root_ref[...], preferred_element_type=jnp.float32)
    for r in range(_R):
        inv = 1.0 / jnp.maximum(cnt_ref[r, 0, 0, :], 1.0)
        acc += jnp.dot(s_ref[r] * inv[:, None], w_ref[r],
                       preferred_element_type=jnp.float32)
    acc += b_ref[...]
    o_ref[...] = jnp.maximum(acc, 0.0) if relu else acc


def _dense_phase(s3, cnt2, x, w, root, b, relu):
    grid = (_N // _C,)
    return pl.pallas_call(
        functools.partial(_dense_body, relu=relu),
        grid=grid,
        in_specs=[
            pl.BlockSpec((_R, _C, _D), lambda i: (0, i, 0)),
            pl.BlockSpec((_R, 1, 1, _C), lambda i: (0, i, 0, 0)),
            pl.BlockSpec((_C, _D), lambda i: (i, 0)),
            pl.BlockSpec((_R, _D, _D), lambda i: (0, 0, 0)),
            pl.BlockSpec((_D, _D), lambda i: (0, 0)),
            pl.BlockSpec((1, _D), lambda i: (0, 0)),
        ],
        out_specs=pl.BlockSpec((_C, _D), lambda i: (i, 0)),
        out_shape=jax.ShapeDtypeStruct((_N, _D), jnp.float32),
    )(s3, cnt2.reshape(_R, _N // _C, 1, _C), x, w, root, b.reshape(1, _D))


def _sc_agg_body(xtab, srcs, segs, s_out,
                 acc, src0, src1, seg0, seg1, idx0, idx1, rows0, rows1,
                 sem0, sem1):
    c = lax.axis_index("c")
    s = lax.axis_index("s")
    cbase = c * _H
    trash_v = _H + s * 8 + (lax.iota(jnp.int32, 16) & 7)

    def _zrow(i, _):
        rows0[i, 0:16] = jnp.zeros((16,), jnp.float32)
        rows0[i, 16:32] = jnp.zeros((16,), jnp.float32)
        return 0

    def _lg(i, src_b, seg_b, rows_b, sem):
        off = s * _EPT + i * _CHUNK
        pltpu.sync_copy(srcs.at[pl.ds(off, _CHUNK)], src_b)
        pltpu.sync_copy(segs.at[pl.ds(off, _CHUNK)], seg_b)
        return pltpu.async_copy(xtab.at[src_b], rows_b, sem)

    def _mkidx(seg_b, idx_b, r):
        base = r * _N + cbase

        def _j(j, r):
            def _v(kk, r):
                loc = seg_b[pl.ds((j * 8 + kk) * 16, 16)] - base
                m = (loc >= 0) & (loc < _H)
                idx_b[j, pl.ds(kk * 16, 16)] = jnp.where(m, loc, trash_v)
                return r
            return lax.fori_loop(0, 8, _v, r)
        lax.fori_loop(0, _CHUNK // 128, _j, r)

    def _scat(rows_b, idx_b, r):
        def _j(j, r):
            pltpu.sync_copy(rows_b.at[pl.ds(j * 128, 128)],
                            acc.at[idx_b.at[j]], add=True)
            return r
        lax.fori_loop(0, _CHUNK // 128, _j, r)

    def _pair(p, r):
        d0 = _lg(2 * p, src0, seg0, rows0, sem0)
        d1 = _lg(2 * p + 1, src1, seg1, rows1, sem1)
        _mkidx(seg0, idx0, r)
        _mkidx(seg1, idx1, r)
        d0.wait()
        _scat(rows0, idx0, r)
        d1.wait()
        _scat(rows1, idx1, r)
        return r

    zbase = s * _TPH
    for r in range(_R):
        lax.fori_loop(0, _CHUNK, _zrow, 0)
        for k in range(12):
            pltpu.sync_copy(rows0, acc.at[pl.ds(zbase + k * _CHUNK, _CHUNK)])

        @pl.when(s < _NTILES - 1)
        def _():
            pltpu.sync_copy(rows0.at[pl.ds(0, 56)],
                            acc.at[pl.ds(zbase + 12 * _CHUNK, 56)])

        @pl.when(s == _NTILES - 1)
        def _():
            pltpu.sync_copy(rows0.at[pl.ds(0, 136)],
                            acc.at[pl.ds(zbase + 12 * _CHUNK, 136)])

        plsc.subcore_barrier()
        lax.fori_loop(0, _EPT // (2 * _CHUNK), _pair, jnp.int32(r))
        plsc.subcore_barrier()

        @pl.when(s < _NTILES - 1)
        def _():
            pltpu.sync_copy(acc.at[pl.ds(zbase, _TPH)],
                            s_out.at[pl.ds(r * _N + cbase + zbase, _TPH)])

        @pl.when(s == _NTILES - 1)
        def _():
            pltpu.sync_copy(acc.at[pl.ds(zbase, 3080)],
                            s_out.at[pl.ds(r * _N + cbase + zbase, 3080)])


def _sc_aggregate(xtab, srcs, segs):
    mesh = plsc.VectorSubcoreMesh(core_axis_name="c", subcore_axis_name="s")
    return pl.kernel(
        _sc_agg_body,
        out_type=jax.ShapeDtypeStruct((_R * _N, _D), jnp.float32),
        mesh=mesh,
        compiler_params=pltpu.CompilerParams(use_tc_tiling_on_sc=False),
        scratch_types=[
            pltpu.VMEM_SHARED((_H + 8 * _NTILES, _D), jnp.float32),
            pltpu.VMEM((_CHUNK,), jnp.int32),
            pltpu.VMEM((_CHUNK,), jnp.int32),
            pltpu.VMEM((_CHUNK,), jnp.int32),
            pltpu.VMEM((_CHUNK,), jnp.int32),
            pltpu.VMEM((_CHUNK // 128, 128), jnp.int32),
            pltpu.VMEM((_CHUNK // 128, 128), jnp.int32),
            pltpu.VMEM((_CHUNK, _D), jnp.float32),
            pltpu.VMEM((_CHUNK, _D), jnp.float32),
            pltpu.SemaphoreType.DMA,
            pltpu.SemaphoreType.DMA,
        ],
    )(xtab, srcs, segs)


def kernel(x, edge_index, edge_type, W1, root1, b1, W2, root2, b2):
    src = edge_index[0]
    dst = edge_index[1]
    seg = edge_type * _N + dst

    cnt = jax.ops.segment_sum(jnp.ones((src.shape[0],), jnp.float32), seg,
                              num_segments=_N * _R)
    cnt2 = cnt.reshape(_R, _N)

    pad = _EPAD - _E
    srcs = jnp.concatenate([src, jnp.zeros((pad,), jnp.int32)])
    segs = jnp.concatenate([seg, jnp.full((pad,), _R * _N, jnp.int32)])

    h = x
    for (w, root, b, relu) in ((W1, root1, b1, True), (W2, root2, b2, False)):
        s = _sc_aggregate(h, srcs, segs)
        h = _dense_phase(s.reshape(_R, _N, _D), cnt2, h, w, root, b, relu)
    return h
